# Initial kernel scaffold; baseline (speedup 1.0000x reference)
import jax
import jax.numpy as jnp
from jax.experimental import pallas as pl


def kernel(loc_preds, conf_preds, priors, target_boxes, target_classes):
    raise NotImplementedError("write your pallas kernel here")


# speedup vs baseline: 6.6712x; 6.6712x over previous
import functools

import jax
import jax.numpy as jnp
from jax import lax
from jax.experimental import pallas as pl

_NUM_CLASSES = 21
_VAR0, _VAR1 = 0.1, 0.2
_THRESH = 0.45
_P = 8732
_T = 8
_LANES = 128
_P_PAD = ((_P + _LANES - 1) // _LANES) * _LANES
_BIG = 2**30


def _mbl_kernel(tb_ref, tc_ref, pri_ref, loc_ref, conf_ref, out_ref):
    b = pl.program_id(0)
    nb = pl.num_programs(0)

    tbb = tb_ref[0]
    tx1 = tbb[:, 0:1]
    ty1 = tbb[:, 1:2]
    tx2 = tbb[:, 2:3]
    ty2 = tbb[:, 3:4]
    labels = tc_ref[0][:, 0:1]

    pri = pri_ref[...]
    pcx = pri[0:1, :]
    pcy = pri[1:2, :]
    pw = pri[2:3, :]
    ph = pri[3:4, :]
    px1 = pcx - pw * 0.5
    py1 = pcy - ph * 0.5
    px2 = pcx + pw * 0.5
    py2 = pcy + ph * 0.5

    iw = jnp.maximum(jnp.minimum(tx2, px2) - jnp.maximum(tx1, px1), 0.0)
    ih = jnp.maximum(jnp.minimum(ty2, py2) - jnp.maximum(ty1, py1), 0.0)
    inter = iw * ih
    area_t = (tx2 - tx1) * (ty2 - ty1)
    area_p = (px2 - px1) * (py2 - py1)
    ov = inter / (area_t + area_p - inter)

    iota_p = lax.broadcasted_iota(jnp.int32, (1, _P_PAD), 1)
    iota_t = lax.broadcasted_iota(jnp.int32, (_T, 1), 0)

    bp_max = jnp.max(ov, axis=1, keepdims=True)
    bp_idx = jnp.min(jnp.where(ov == bp_max, iota_p, _BIG),
                     axis=1, keepdims=True)
    bt_ov = jnp.max(ov, axis=0, keepdims=True)
    bt_max = bt_ov
    bt_idx = jnp.min(jnp.where(ov == bt_max, iota_t, _BIG),
                     axis=0, keepdims=True)

    eq = bp_idx == iota_p
    forced = jnp.max(jnp.where(eq, iota_t, -1), axis=0, keepdims=True)
    bt_ov = jnp.where(forced >= 0, 2.0, bt_ov)
    bt_idx = jnp.where(forced >= 0, forced, bt_idx)

    eqb = bt_idx == iota_t
    zf = jnp.zeros_like(ov)
    m_x1 = jnp.sum(jnp.where(eqb, tx1 + zf, zf), axis=0, keepdims=True)
    m_y1 = jnp.sum(jnp.where(eqb, ty1 + zf, zf), axis=0, keepdims=True)
    m_x2 = jnp.sum(jnp.where(eqb, tx2 + zf, zf), axis=0, keepdims=True)
    m_y2 = jnp.sum(jnp.where(eqb, ty2 + zf, zf), axis=0, keepdims=True)
    zi = jnp.zeros((_T, _P_PAD), jnp.int32)
    m_lab = jnp.sum(jnp.where(eqb, labels + zi, zi), axis=0, keepdims=True)

    conf_cls = jnp.where(bt_ov < _THRESH, 0, m_lab + 1)
    pos = conf_cls > 0
    posf = pos.astype(jnp.float32)
    num_pos = jnp.sum(conf_cls > 0)

    g_cx = ((m_x1 + m_x2) * 0.5 - pcx) / (_VAR0 * pw)
    g_cy = ((m_y1 + m_y2) * 0.5 - pcy) / (_VAR0 * ph)
    g_w = jnp.log((m_x2 - m_x1) / pw) / _VAR1
    g_h = jnp.log((m_y2 - m_y1) / ph) / _VAR1
    lp = loc_ref[0]
    loss_l = jnp.float32(0.0)
    for i, g in enumerate((g_cx, g_cy, g_w, g_h)):
        d = jnp.abs(lp[i:i + 1, :] - g)
        sl1 = jnp.where(d < 1.0, 0.5 * d * d, d - 0.5)
        loss_l = loss_l + jnp.sum(sl1 * posf)

    x = conf_ref[0]
    xm = jnp.max(x, axis=0, keepdims=True)
    s = jnp.sum(jnp.exp(x - xm), axis=0, keepdims=True)
    lse = xm + jnp.log(s)
    iota_c = lax.broadcasted_iota(jnp.int32, (_NUM_CLASSES, 1), 0)
    eqc = conf_cls == iota_c
    x_sel = jnp.sum(jnp.where(eqc, x, 0.0), axis=0, keepdims=True)
    ce = lse - x_sel
    sum_pos_ce = jnp.sum(ce * posf)

    valid = iota_p < _P
    loss_gt = jnp.where(jnp.logical_and(valid, jnp.logical_not(pos)),
                        lse - x[0:1, :], 0.0)

    k = jnp.minimum(3 * num_pos, _P - 1)

    def bs_body(_, carry):
        lo, hi = carry
        mid = lo + (hi - lo) // 2
        mid_f = lax.bitcast_convert_type(mid, jnp.float32)
        cnt = jnp.sum((loss_gt > mid_f).astype(jnp.int32))
        below = cnt < k
        return (jnp.where(below, lo, mid + 1), jnp.where(below, mid, hi))

    lo0 = jnp.int32(0)
    hi0 = jnp.int32(0x7F800000)
    _, t_bits = lax.fori_loop(0, 31, bs_body, (lo0, hi0))
    t = lax.bitcast_convert_type(t_bits, jnp.float32)
    gt = loss_gt > t
    cnt_gt = jnp.sum(gt.astype(jnp.int32))
    sum_topk = (jnp.sum(jnp.where(gt, loss_gt, 0.0))
                + (k - cnt_gt).astype(jnp.float32) * t)
    loss_c = sum_pos_ce + sum_topk

    r = lax.broadcasted_iota(jnp.int32, (8, _LANES), 0)
    c = lax.broadcasted_iota(jnp.int32, (8, _LANES), 1)
    contrib = jnp.where((r == 0) & (c == 0), loss_l, 0.0)
    contrib = contrib + jnp.where((r == 0) & (c == 1), loss_c, 0.0)
    contrib = contrib + jnp.where((r == 0) & (c == 2),
                                  num_pos.astype(jnp.float32), 0.0)

    @pl.when(b == 0)
    def _():
        out_ref[...] = contrib

    @pl.when(b > 0)
    def _():
        out_ref[...] = out_ref[...] + contrib
    del nb


@functools.partial(jax.jit, static_argnames=("interpret",))
def kernel(loc_preds, conf_preds, priors, target_boxes, target_classes,
           interpret=False):
    B, P, C = conf_preds.shape
    pad = _P_PAD - P

    tb = jnp.zeros((B, _T, _LANES), jnp.float32).at[:, :, :4].set(target_boxes)
    tc = jnp.zeros((B, _T, _LANES), jnp.int32).at[:, :, 0:1].set(
        target_classes.astype(jnp.int32)[..., None])
    pri_t = jnp.transpose(priors, (1, 0))
    pri_pad = jnp.tile(jnp.array([[-10.0], [-10.0], [1.0], [1.0]],
                                 jnp.float32), (1, pad))
    pri_t = jnp.concatenate([pri_t, pri_pad], axis=1)
    loc_t = jnp.pad(jnp.transpose(loc_preds, (0, 2, 1)),
                    ((0, 0), (0, 0), (0, pad)))
    conf_t = jnp.pad(jnp.transpose(conf_preds, (0, 2, 1)),
                     ((0, 0), (0, 0), (0, pad)))

    out = pl.pallas_call(
        _mbl_kernel,
        grid=(B,),
        in_specs=[
            pl.BlockSpec((1, _T, _LANES), lambda b: (b, 0, 0)),
            pl.BlockSpec((1, _T, _LANES), lambda b: (b, 0, 0)),
            pl.BlockSpec((4, _P_PAD), lambda b: (0, 0)),
            pl.BlockSpec((1, 4, _P_PAD), lambda b: (b, 0, 0)),
            pl.BlockSpec((1, C, _P_PAD), lambda b: (b, 0, 0)),
        ],
        out_specs=pl.BlockSpec((8, _LANES), lambda b: (0, 0)),
        out_shape=jax.ShapeDtypeStruct((8, _LANES), jnp.float32),
        interpret=interpret,
    )(tb, tc, pri_t, loc_t, conf_t)

    n = out[0, 2]
    return out[0, 0] / n, out[0, 1] / n


# speedup vs baseline: 12.1434x; 1.8203x over previous
import functools

import jax
import jax.numpy as jnp
from jax import lax
from jax.experimental import pallas as pl
from jax.experimental.pallas import tpu as pltpu

_NUM_CLASSES = 21
_VAR0, _VAR1 = 0.1, 0.2
_THRESH = 0.45
_P = 8732
_T = 8
_B = 16
_LANES = 128
_P_PAD = ((_P + _LANES - 1) // _LANES) * _LANES
_BIG = 2**30


def _mbl_kernel(tb_ref, tc_ref, pri_ref, loc_ref, conf_ref, out_ref,
                lg_ref, np_ref):
    b = pl.program_id(0)
    nb = pl.num_programs(0)

    tbb = tb_ref[0]
    tx1 = tbb[:, 0:1]
    ty1 = tbb[:, 1:2]
    tx2 = tbb[:, 2:3]
    ty2 = tbb[:, 3:4]
    labels = tc_ref[0][:, 0:1]

    pri = pri_ref[...]
    pcx = pri[0:1, :]
    pcy = pri[1:2, :]
    pw = pri[2:3, :]
    ph = pri[3:4, :]
    px1 = pcx - pw * 0.5
    py1 = pcy - ph * 0.5
    px2 = pcx + pw * 0.5
    py2 = pcy + ph * 0.5

    iw = jnp.maximum(jnp.minimum(tx2, px2) - jnp.maximum(tx1, px1), 0.0)
    ih = jnp.maximum(jnp.minimum(ty2, py2) - jnp.maximum(ty1, py1), 0.0)
    inter = iw * ih
    area_t = (tx2 - tx1) * (ty2 - ty1)
    area_p = (px2 - px1) * (py2 - py1)
    ov = inter / (area_t + area_p - inter)

    iota_p = lax.broadcasted_iota(jnp.int32, (1, _P_PAD), 1)
    iota_t = lax.broadcasted_iota(jnp.int32, (_T, 1), 0)

    bp_max = jnp.max(ov, axis=1, keepdims=True)
    bp_idx = jnp.min(jnp.where(ov == bp_max, iota_p, _BIG),
                     axis=1, keepdims=True)
    bt_ov = jnp.max(ov, axis=0, keepdims=True)
    bt_idx = jnp.min(jnp.where(ov == bt_ov, iota_t, _BIG),
                     axis=0, keepdims=True)

    eq = bp_idx == iota_p
    forced = jnp.max(jnp.where(eq, iota_t, -1), axis=0, keepdims=True)
    bt_ov = jnp.where(forced >= 0, 2.0, bt_ov)
    bt_idx = jnp.where(forced >= 0, forced, bt_idx)

    eqb = bt_idx == iota_t
    zf = jnp.zeros_like(ov)
    m_x1 = jnp.sum(jnp.where(eqb, tx1 + zf, zf), axis=0, keepdims=True)
    m_y1 = jnp.sum(jnp.where(eqb, ty1 + zf, zf), axis=0, keepdims=True)
    m_x2 = jnp.sum(jnp.where(eqb, tx2 + zf, zf), axis=0, keepdims=True)
    m_y2 = jnp.sum(jnp.where(eqb, ty2 + zf, zf), axis=0, keepdims=True)
    zi = jnp.zeros((_T, _P_PAD), jnp.int32)
    m_lab = jnp.sum(jnp.where(eqb, labels + zi, zi), axis=0, keepdims=True)

    conf_cls = jnp.where(bt_ov < _THRESH, 0, m_lab + 1)
    pos = conf_cls > 0
    posf = pos.astype(jnp.float32)
    num_pos = jnp.sum(conf_cls > 0)

    g_cx = ((m_x1 + m_x2) * 0.5 - pcx) / (_VAR0 * pw)
    g_cy = ((m_y1 + m_y2) * 0.5 - pcy) / (_VAR0 * ph)
    g_w = jnp.log((m_x2 - m_x1) / pw) / _VAR1
    g_h = jnp.log((m_y2 - m_y1) / ph) / _VAR1
    lp = loc_ref[0]
    loss_l = jnp.float32(0.0)
    for i, g in enumerate((g_cx, g_cy, g_w, g_h)):
        d = jnp.abs(lp[i:i + 1, :] - g)
        sl1 = jnp.where(d < 1.0, 0.5 * d * d, d - 0.5)
        loss_l = loss_l + jnp.sum(sl1 * posf)

    x = conf_ref[0]
    xm = jnp.max(x, axis=0, keepdims=True)
    s = jnp.sum(jnp.exp(x - xm), axis=0, keepdims=True)
    lse = xm + jnp.log(s)
    iota_c = lax.broadcasted_iota(jnp.int32, (_NUM_CLASSES, 1), 0)
    eqc = conf_cls == iota_c
    x_sel = jnp.sum(jnp.where(eqc, x, 0.0), axis=0, keepdims=True)
    ce = lse - x_sel
    sum_pos_ce = jnp.sum(ce * posf)

    valid = iota_p < _P
    loss_gt = jnp.where(jnp.logical_and(valid, jnp.logical_not(pos)),
                        lse - x[0:1, :], 0.0)

    row16 = lax.broadcasted_iota(jnp.int32, (_B, 1), 0)
    bmask = row16 == b
    lg_ref[...] = jnp.where(bmask, loss_gt, lg_ref[...])
    np_ref[...] = jnp.where(bmask, num_pos, np_ref[...])

    r = lax.broadcasted_iota(jnp.int32, (8, _LANES), 0)
    c = lax.broadcasted_iota(jnp.int32, (8, _LANES), 1)
    contrib = jnp.where((r == 0) & (c == 0), loss_l, 0.0)
    contrib = contrib + jnp.where((r == 0) & (c == 1), sum_pos_ce, 0.0)
    contrib = contrib + jnp.where((r == 0) & (c == 2),
                                  num_pos.astype(jnp.float32), 0.0)

    @pl.when(b == 0)
    def _():
        out_ref[...] = contrib

    @pl.when(b > 0)
    def _():
        out_ref[...] = out_ref[...] + contrib

    @pl.when(b == nb - 1)
    def _():
        lg = lg_ref[...]
        kk = jnp.minimum(3 * np_ref[...][:, 0:1], _P - 1)

        def bs_body(_, carry):
            lo, hi = carry
            mid = lo + (hi - lo) // 2
            mid_f = lax.bitcast_convert_type(mid, jnp.float32)
            cnt = jnp.sum((lg > mid_f).astype(jnp.int32),
                          axis=1, keepdims=True)
            below = cnt < kk
            return (jnp.where(below, lo, mid + 1),
                    jnp.where(below, mid, hi))

        lo0 = jnp.zeros((_B, 1), jnp.int32)
        hi0 = jnp.full((_B, 1), 0x7F800000, jnp.int32)
        _, t_bits = lax.fori_loop(0, 31, bs_body, (lo0, hi0))
        t = lax.bitcast_convert_type(t_bits, jnp.float32)
        gt = lg > t
        cnt_gt = jnp.sum(gt.astype(jnp.int32), axis=1, keepdims=True)
        rows = (jnp.sum(jnp.where(gt, lg, 0.0), axis=1, keepdims=True)
                + (kk - cnt_gt).astype(jnp.float32) * t)
        rows = jnp.where(kk > 0, rows, 0.0)
        topk_total = jnp.sum(rows)
        out_ref[...] = out_ref[...] + jnp.where((r == 0) & (c == 1),
                                                topk_total, 0.0)


@functools.partial(jax.jit, static_argnames=("interpret",))
def kernel(loc_preds, conf_preds, priors, target_boxes, target_classes,
           interpret=False):
    B, P, C = conf_preds.shape
    pad = _P_PAD - P

    tb = jnp.zeros((B, _T, _LANES), jnp.float32).at[:, :, :4].set(target_boxes)
    tc = jnp.zeros((B, _T, _LANES), jnp.int32).at[:, :, 0:1].set(
        target_classes.astype(jnp.int32)[..., None])
    pri_t = jnp.transpose(priors, (1, 0))
    pri_pad = jnp.tile(jnp.array([[-10.0], [-10.0], [1.0], [1.0]],
                                 jnp.float32), (1, pad))
    pri_t = jnp.concatenate([pri_t, pri_pad], axis=1)
    loc_t = jnp.pad(jnp.transpose(loc_preds, (0, 2, 1)),
                    ((0, 0), (0, 0), (0, pad)))
    conf_t = jnp.pad(jnp.transpose(conf_preds, (0, 2, 1)),
                     ((0, 0), (0, 0), (0, pad)))

    out = pl.pallas_call(
        _mbl_kernel,
        grid=(B,),
        in_specs=[
            pl.BlockSpec((1, _T, _LANES), lambda b: (b, 0, 0)),
            pl.BlockSpec((1, _T, _LANES), lambda b: (b, 0, 0)),
            pl.BlockSpec((4, _P_PAD), lambda b: (0, 0)),
            pl.BlockSpec((1, 4, _P_PAD), lambda b: (b, 0, 0)),
            pl.BlockSpec((1, C, _P_PAD), lambda b: (b, 0, 0)),
        ],
        out_specs=pl.BlockSpec((8, _LANES), lambda b: (0, 0)),
        out_shape=jax.ShapeDtypeStruct((8, _LANES), jnp.float32),
        scratch_shapes=[
            pltpu.VMEM((_B, _P_PAD), jnp.float32),
            pltpu.VMEM((_B, _LANES), jnp.int32),
        ],
        interpret=interpret,
    )(tb, tc, pri_t, loc_t, conf_t)

    n = out[0, 2]
    return out[0, 0] / n, out[0, 1] / n


# speedup vs baseline: 12.6160x; 1.0389x over previous
import functools

import jax
import jax.numpy as jnp
from jax import lax
from jax.experimental import pallas as pl
from jax.experimental.pallas import tpu as pltpu

_NUM_CLASSES = 21
_VAR0, _VAR1 = 0.1, 0.2
_THRESH = 0.45
_P = 8732
_T = 8
_B = 16
_LANES = 128
_P_PAD = ((_P + _LANES - 1) // _LANES) * _LANES
_BIG = 2**30
_BPS = 2


def _mbl_kernel(tb_ref, tc_ref, pri_ref, loc_ref, conf_ref, out_ref,
                lg_ref, np_ref):
    b = pl.program_id(0)
    nb = pl.num_programs(0)

    total_contrib = jnp.zeros((8, _LANES), jnp.float32)
    for u in range(_BPS):
        total_contrib = total_contrib + _one_batch(
            _BPS * b + u, u, tb_ref, tc_ref, pri_ref, loc_ref, conf_ref,
            lg_ref, np_ref)

    @pl.when(b == 0)
    def _():
        out_ref[...] = total_contrib

    @pl.when(b > 0)
    def _():
        out_ref[...] = out_ref[...] + total_contrib

    @pl.when(b == nb - 1)
    def _():
        r = lax.broadcasted_iota(jnp.int32, (8, _LANES), 0)
        c = lax.broadcasted_iota(jnp.int32, (8, _LANES), 1)
        lg = lg_ref[...]
        kk = jnp.minimum(3 * np_ref[...][:, 0:1], _P - 1)

        def bs_body(_, carry):
            lo, hi = carry
            mid = lo + (hi - lo) // 2
            mid_f = lax.bitcast_convert_type(mid, jnp.float32)
            cnt = jnp.sum((lg > mid_f).astype(jnp.int32),
                          axis=1, keepdims=True)
            below = cnt < kk
            return (jnp.where(below, lo, mid + 1),
                    jnp.where(below, mid, hi))

        lo0 = jnp.zeros((_B, 1), jnp.int32)
        hi0 = jnp.full((_B, 1), 0x7F800000, jnp.int32)
        _, t_bits = lax.fori_loop(0, 31, bs_body, (lo0, hi0))
        t = lax.bitcast_convert_type(t_bits, jnp.float32)
        gt = lg > t
        cnt_gt = jnp.sum(gt.astype(jnp.int32), axis=1, keepdims=True)
        rows = (jnp.sum(jnp.where(gt, lg, 0.0), axis=1, keepdims=True)
                + (kk - cnt_gt).astype(jnp.float32) * t)
        rows = jnp.where(kk > 0, rows, 0.0)
        topk_total = jnp.sum(rows)
        out_ref[...] = out_ref[...] + jnp.where((r == 0) & (c == 1),
                                                topk_total, 0.0)


def _one_batch(bb, u, tb_ref, tc_ref, pri_ref, loc_ref, conf_ref,
               lg_ref, np_ref):
    tbb = tb_ref[u]
    tx1 = tbb[:, 0:1]
    ty1 = tbb[:, 1:2]
    tx2 = tbb[:, 2:3]
    ty2 = tbb[:, 3:4]
    labels = tc_ref[u][:, 0:1]

    pri = pri_ref[...]
    pcx = pri[0:1, :]
    pcy = pri[1:2, :]
    pw = pri[2:3, :]
    ph = pri[3:4, :]
    px1 = pcx - pw * 0.5
    py1 = pcy - ph * 0.5
    px2 = pcx + pw * 0.5
    py2 = pcy + ph * 0.5

    iw = jnp.maximum(jnp.minimum(tx2, px2) - jnp.maximum(tx1, px1), 0.0)
    ih = jnp.maximum(jnp.minimum(ty2, py2) - jnp.maximum(ty1, py1), 0.0)
    inter = iw * ih
    area_t = (tx2 - tx1) * (ty2 - ty1)
    area_p = (px2 - px1) * (py2 - py1)
    ov = inter / (area_t + area_p - inter)

    iota_p = lax.broadcasted_iota(jnp.int32, (1, _P_PAD), 1)
    iota_t = lax.broadcasted_iota(jnp.int32, (_T, 1), 0)

    bp_max = jnp.max(ov, axis=1, keepdims=True)
    bp_idx = jnp.min(jnp.where(ov == bp_max, iota_p, _BIG),
                     axis=1, keepdims=True)
    bt_ov = jnp.max(ov, axis=0, keepdims=True)
    bt_idx = jnp.min(jnp.where(ov == bt_ov, iota_t, _BIG),
                     axis=0, keepdims=True)

    eq = bp_idx == iota_p
    forced = jnp.max(jnp.where(eq, iota_t, -1), axis=0, keepdims=True)
    bt_ov = jnp.where(forced >= 0, 2.0, bt_ov)
    bt_idx = jnp.where(forced >= 0, forced, bt_idx)

    eqb = bt_idx == iota_t
    zf = jnp.zeros_like(ov)
    m_x1 = jnp.sum(jnp.where(eqb, tx1 + zf, zf), axis=0, keepdims=True)
    m_y1 = jnp.sum(jnp.where(eqb, ty1 + zf, zf), axis=0, keepdims=True)
    m_x2 = jnp.sum(jnp.where(eqb, tx2 + zf, zf), axis=0, keepdims=True)
    m_y2 = jnp.sum(jnp.where(eqb, ty2 + zf, zf), axis=0, keepdims=True)
    zi = jnp.zeros((_T, _P_PAD), jnp.int32)
    m_lab = jnp.sum(jnp.where(eqb, labels + zi, zi), axis=0, keepdims=True)

    conf_cls = jnp.where(bt_ov < _THRESH, 0, m_lab + 1)
    pos = conf_cls > 0
    posf = pos.astype(jnp.float32)
    num_pos = jnp.sum(conf_cls > 0)

    g_cx = ((m_x1 + m_x2) * 0.5 - pcx) / (_VAR0 * pw)
    g_cy = ((m_y1 + m_y2) * 0.5 - pcy) / (_VAR0 * ph)
    g_w = jnp.log((m_x2 - m_x1) / pw) / _VAR1
    g_h = jnp.log((m_y2 - m_y1) / ph) / _VAR1
    lp = loc_ref[u]
    loss_l = jnp.float32(0.0)
    for i, g in enumerate((g_cx, g_cy, g_w, g_h)):
        d = jnp.abs(lp[i:i + 1, :] - g)
        sl1 = jnp.where(d < 1.0, 0.5 * d * d, d - 0.5)
        loss_l = loss_l + jnp.sum(sl1 * posf)

    x = conf_ref[u]
    xm = jnp.max(x, axis=0, keepdims=True)
    s = jnp.sum(jnp.exp(x - xm), axis=0, keepdims=True)
    lse = xm + jnp.log(s)
    iota_c = lax.broadcasted_iota(jnp.int32, (_NUM_CLASSES, 1), 0)
    eqc = conf_cls == iota_c
    x_sel = jnp.sum(jnp.where(eqc, x, 0.0), axis=0, keepdims=True)
    ce = lse - x_sel
    sum_pos_ce = jnp.sum(ce * posf)

    valid = iota_p < _P
    loss_gt = jnp.where(jnp.logical_and(valid, jnp.logical_not(pos)),
                        lse - x[0:1, :], 0.0)

    row16 = lax.broadcasted_iota(jnp.int32, (_B, 1), 0)
    bmask = row16 == bb
    lg_ref[...] = jnp.where(bmask, loss_gt, lg_ref[...])
    np_ref[...] = jnp.where(bmask, num_pos, np_ref[...])

    r = lax.broadcasted_iota(jnp.int32, (8, _LANES), 0)
    c = lax.broadcasted_iota(jnp.int32, (8, _LANES), 1)
    contrib = jnp.where((r == 0) & (c == 0), loss_l, 0.0)
    contrib = contrib + jnp.where((r == 0) & (c == 1), sum_pos_ce, 0.0)
    contrib = contrib + jnp.where((r == 0) & (c == 2),
                                  num_pos.astype(jnp.float32), 0.0)
    return contrib


@functools.partial(jax.jit, static_argnames=("interpret",))
def kernel(loc_preds, conf_preds, priors, target_boxes, target_classes,
           interpret=False):
    B, P, C = conf_preds.shape
    pad = _P_PAD - P

    tb = jnp.zeros((B, _T, _LANES), jnp.float32).at[:, :, :4].set(target_boxes)
    tc = jnp.zeros((B, _T, _LANES), jnp.int32).at[:, :, 0:1].set(
        target_classes.astype(jnp.int32)[..., None])
    pri_t = jnp.transpose(priors, (1, 0))
    pri_pad = jnp.tile(jnp.array([[-10.0], [-10.0], [1.0], [1.0]],
                                 jnp.float32), (1, pad))
    pri_t = jnp.concatenate([pri_t, pri_pad], axis=1)
    loc_t = jnp.pad(jnp.transpose(loc_preds, (0, 2, 1)),
                    ((0, 0), (0, 0), (0, pad)))
    conf_t = jnp.pad(jnp.transpose(conf_preds, (0, 2, 1)),
                     ((0, 0), (0, 0), (0, pad)))

    out = pl.pallas_call(
        _mbl_kernel,
        grid=(B // _BPS,),
        in_specs=[
            pl.BlockSpec((_BPS, _T, _LANES), lambda b: (b, 0, 0)),
            pl.BlockSpec((_BPS, _T, _LANES), lambda b: (b, 0, 0)),
            pl.BlockSpec((4, _P_PAD), lambda b: (0, 0)),
            pl.BlockSpec((_BPS, 4, _P_PAD), lambda b: (b, 0, 0)),
            pl.BlockSpec((_BPS, C, _P_PAD), lambda b: (b, 0, 0)),
        ],
        out_specs=pl.BlockSpec((8, _LANES), lambda b: (0, 0)),
        out_shape=jax.ShapeDtypeStruct((8, _LANES), jnp.float32),
        scratch_shapes=[
            pltpu.VMEM((_B, _P_PAD), jnp.float32),
            pltpu.VMEM((_B, _LANES), jnp.int32),
        ],
        interpret=interpret,
    )(tb, tc, pri_t, loc_t, conf_t)

    n = out[0, 2]
    return out[0, 0] / n, out[0, 1] / n


# speedup vs baseline: 15.2736x; 1.2107x over previous
import functools

import jax
import jax.numpy as jnp
from jax import lax
from jax.experimental import pallas as pl
from jax.experimental.pallas import tpu as pltpu

_NUM_CLASSES = 21
_THRESH = 0.45
_P = 8732
_T = 8
_B = 16
_LANES = 128
_P_PAD = ((_P + _LANES - 1) // _LANES) * _LANES
_BIG = 2**30
_BPS = 2

_DN = (((0,), (0,)), ((), ()))


def _one_batch(bb, u, tb_ref, tc_ref, pri_ref, loc_ref, conf_ref,
               lg_ref, np_ref):
    tbb = tb_ref[u]
    tx1 = tbb[:, 0:1]
    ty1 = tbb[:, 1:2]
    tx2 = tbb[:, 2:3]
    ty2 = tbb[:, 3:4]
    labf = tc_ref[u][:, 0:1]

    tw = tx2 - tx1
    th = ty2 - ty1
    area_t = tw * th
    qt = jnp.concatenate(
        [(tx1 + tx2) * 0.5, (ty1 + ty2) * 0.5,
         5.0 * jnp.log(tw), 5.0 * jnp.log(th), labf], axis=1)

    pp = pri_ref[...]
    px1 = pp[0:1, :]
    py1 = pp[1:2, :]
    px2 = pp[2:3, :]
    py2 = pp[3:4, :]
    area_p = pp[4:5, :]
    pcx = pp[5:6, :]
    pcy = pp[6:7, :]
    inv01w = pp[7:8, :]
    inv01h = pp[8:9, :]
    lpw5 = pp[9:10, :]
    lph5 = pp[10:11, :]

    iw = jnp.maximum(jnp.minimum(tx2, px2) - jnp.maximum(tx1, px1), 0.0)
    ih = jnp.maximum(jnp.minimum(ty2, py2) - jnp.maximum(ty1, py1), 0.0)
    inter = iw * ih
    ov = inter / (area_t + area_p - inter)

    iota_p = lax.broadcasted_iota(jnp.int32, (1, _P_PAD), 1)
    iota_t = lax.broadcasted_iota(jnp.int32, (_T, 1), 0)

    bp_max = jnp.max(ov, axis=1, keepdims=True)
    bp_idx = jnp.min(jnp.where(ov == bp_max, iota_p, _BIG),
                     axis=1, keepdims=True)
    bt_ov = jnp.max(ov, axis=0, keepdims=True)
    bt_idx = jnp.min(jnp.where(ov == bt_ov, iota_t, _BIG),
                     axis=0, keepdims=True)

    eq = bp_idx == iota_p
    forced = jnp.max(jnp.where(eq, iota_t, -1), axis=0, keepdims=True)
    bt_ov = jnp.where(forced >= 0, 2.0, bt_ov)
    bt_idx = jnp.where(forced >= 0, forced, bt_idx)

    onehot = jnp.where(bt_idx == iota_t, 1.0, 0.0)
    m = lax.dot_general(qt, onehot, _DN,
                        preferred_element_type=jnp.float32)
    m_lab = jnp.floor(m[4:5, :] + 0.5)
    conf_f = jnp.where(bt_ov < _THRESH, 0.0, m_lab + 1.0)
    pos = conf_f > 0.5
    posf = pos.astype(jnp.float32)
    num_pos = jnp.sum(posf)

    g_cx = (m[0:1, :] - pcx) * inv01w
    g_cy = (m[1:2, :] - pcy) * inv01h
    g_w = m[2:3, :] - lpw5
    g_h = m[3:4, :] - lph5
    lp = loc_ref[u]
    sl1 = jnp.zeros((1, _P_PAD), jnp.float32)
    for i, g in enumerate((g_cx, g_cy, g_w, g_h)):
        d = jnp.abs(lp[i:i + 1, :] - g)
        sl1 = sl1 + jnp.where(d < 1.0, 0.5 * d * d, d - 0.5)
    loss_l = jnp.sum(sl1 * posf)

    x = conf_ref[u]
    xm = jnp.max(x, axis=0, keepdims=True)
    e = jnp.exp(x - xm)
    ones_r = jnp.ones((1, _NUM_CLASSES), jnp.float32)
    dn_std = (((1,), (0,)), ((), ()))
    s = lax.dot_general(ones_r, e, dn_std,
                        preferred_element_type=jnp.float32)
    lse = xm + jnp.log(s)
    iota_cf = lax.broadcasted_iota(
        jnp.int32, (_NUM_CLASSES, 1), 0).astype(jnp.float32)
    xsel_m = jnp.where(conf_f == iota_cf, x, 0.0)
    x_sel = lax.dot_general(ones_r, xsel_m, dn_std,
                            preferred_element_type=jnp.float32)
    sum_pos_ce = jnp.sum((lse - x_sel) * posf)

    valid = iota_p < _P
    loss_gt = jnp.where(jnp.logical_and(valid, jnp.logical_not(pos)),
                        lse - x[0:1, :], 0.0)

    row16 = lax.broadcasted_iota(jnp.int32, (_B, 1), 0)
    bmask = row16 == bb
    lg_ref[...] = jnp.where(bmask, loss_gt, lg_ref[...])
    np_ref[...] = jnp.where(bmask, num_pos, np_ref[...])

    r = lax.broadcasted_iota(jnp.int32, (8, _LANES), 0)
    c = lax.broadcasted_iota(jnp.int32, (8, _LANES), 1)
    contrib = jnp.where((r == 0) & (c == 0), loss_l, 0.0)
    contrib = contrib + jnp.where((r == 0) & (c == 1), sum_pos_ce, 0.0)
    contrib = contrib + jnp.where((r == 0) & (c == 2), num_pos, 0.0)
    return contrib


def _mbl_kernel(tb_ref, tc_ref, pri_ref, loc_ref, conf_ref, out_ref,
                lg_ref, np_ref):
    b = pl.program_id(0)
    nb = pl.num_programs(0)

    total_contrib = jnp.zeros((8, _LANES), jnp.float32)
    for u in range(_BPS):
        total_contrib = total_contrib + _one_batch(
            _BPS * b + u, u, tb_ref, tc_ref, pri_ref, loc_ref, conf_ref,
            lg_ref, np_ref)

    @pl.when(b == 0)
    def _():
        out_ref[...] = total_contrib

    @pl.when(b > 0)
    def _():
        out_ref[...] = out_ref[...] + total_contrib

    @pl.when(b == nb - 1)
    def _():
        r = lax.broadcasted_iota(jnp.int32, (8, _LANES), 0)
        c = lax.broadcasted_iota(jnp.int32, (8, _LANES), 1)
        lg = lg_ref[...]
        npos = np_ref[...][:, 0:1].astype(jnp.int32)
        kk = jnp.minimum(3 * npos, _P - 1)

        def bs_body(_, carry):
            lo, hi = carry
            mid = lo + (hi - lo) // 2
            mid_f = lax.bitcast_convert_type(mid, jnp.float32)
            cnt = jnp.sum((lg > mid_f).astype(jnp.int32),
                          axis=1, keepdims=True)
            below = cnt < kk
            return (jnp.where(below, lo, mid + 1),
                    jnp.where(below, mid, hi))

        lo0 = jnp.zeros((_B, 1), jnp.int32)
        hi0 = jnp.full((_B, 1), 0x7F800000, jnp.int32)
        _, t_bits = lax.fori_loop(0, 31, bs_body, (lo0, hi0))
        t = lax.bitcast_convert_type(t_bits, jnp.float32)
        gt = lg > t
        cnt_gt = jnp.sum(gt.astype(jnp.int32), axis=1, keepdims=True)
        rows = (jnp.sum(jnp.where(gt, lg, 0.0), axis=1, keepdims=True)
                + (kk - cnt_gt).astype(jnp.float32) * t)
        rows = jnp.where(kk > 0, rows, 0.0)
        topk_total = jnp.sum(rows)
        out_ref[...] = out_ref[...] + jnp.where((r == 0) & (c == 1),
                                                topk_total, 0.0)


@functools.partial(jax.jit, static_argnames=("interpret",))
def kernel(loc_preds, conf_preds, priors, target_boxes, target_classes,
           interpret=False):
    B, P, C = conf_preds.shape
    pad = _P_PAD - P

    tb = jnp.zeros((B, _T, _LANES), jnp.float32).at[:, :, :4].set(target_boxes)
    tc = jnp.zeros((B, _T, _LANES), jnp.float32).at[:, :, 0:1].set(
        target_classes.astype(jnp.float32)[..., None])
    pri_t = jnp.transpose(priors, (1, 0))
    pri_pad = jnp.tile(jnp.array([[-10.0], [-10.0], [1.0], [1.0]],
                                 jnp.float32), (1, pad))
    pri_t = jnp.concatenate([pri_t, pri_pad], axis=1)
    pcx, pcy, pw, ph = (pri_t[i] for i in range(4))
    cx1, cy1 = pcx - pw / 2.0, pcy - ph / 2.0
    cx2, cy2 = pcx + pw / 2.0, pcy + ph / 2.0
    pp = jnp.stack([
        cx1, cy1, cx2, cy2,
        (cx2 - cx1) * (cy2 - cy1), pcx, pcy, 10.0 / pw, 10.0 / ph,
        5.0 * jnp.log(pw), 5.0 * jnp.log(ph),
    ])
    pp = jnp.concatenate(
        [pp, jnp.zeros((5, _P_PAD), jnp.float32)], axis=0)
    loc_t = jnp.pad(jnp.transpose(loc_preds, (0, 2, 1)),
                    ((0, 0), (0, 0), (0, pad)))
    conf_t = jnp.pad(jnp.transpose(conf_preds, (0, 2, 1)),
                     ((0, 0), (0, 0), (0, pad)))

    out = pl.pallas_call(
        _mbl_kernel,
        grid=(B // _BPS,),
        in_specs=[
            pl.BlockSpec((_BPS, _T, _LANES), lambda b: (b, 0, 0)),
            pl.BlockSpec((_BPS, _T, _LANES), lambda b: (b, 0, 0)),
            pl.BlockSpec((16, _P_PAD), lambda b: (0, 0)),
            pl.BlockSpec((_BPS, 4, _P_PAD), lambda b: (b, 0, 0)),
            pl.BlockSpec((_BPS, C, _P_PAD), lambda b: (b, 0, 0)),
        ],
        out_specs=pl.BlockSpec((8, _LANES), lambda b: (0, 0)),
        out_shape=jax.ShapeDtypeStruct((8, _LANES), jnp.float32),
        scratch_shapes=[
            pltpu.VMEM((_B, _P_PAD), jnp.float32),
            pltpu.VMEM((_B, _LANES), jnp.float32),
        ],
        interpret=interpret,
    )(tb, tc, pp, loc_t, conf_t)

    n = out[0, 2]
    return out[0, 0] / n, out[0, 1] / n


# speedup vs baseline: 15.5006x; 1.0149x over previous
import functools

import jax
import jax.numpy as jnp
from jax import lax
from jax.experimental import pallas as pl
from jax.experimental.pallas import tpu as pltpu

_NUM_CLASSES = 21
_THRESH = 0.45
_P = 8732
_T = 8
_B = 16
_LANES = 128
_P_PAD = ((_P + _LANES - 1) // _LANES) * _LANES
_BIG = 2**30
_BPS = 4

_DN = (((0,), (0,)), ((), ()))


def _one_batch(bb, u, tb_ref, tc_ref, pri_ref, loc_ref, conf_ref,
               lg_ref, np_ref):
    tbb = tb_ref[u]
    tx1 = tbb[:, 0:1]
    ty1 = tbb[:, 1:2]
    tx2 = tbb[:, 2:3]
    ty2 = tbb[:, 3:4]
    labf = tc_ref[u][:, 0:1]

    tw = tx2 - tx1
    th = ty2 - ty1
    area_t = tw * th
    qt = jnp.concatenate(
        [(tx1 + tx2) * 0.5, (ty1 + ty2) * 0.5,
         5.0 * jnp.log(tw), 5.0 * jnp.log(th), labf], axis=1)

    pp = pri_ref[...]
    px1 = pp[0:1, :]
    py1 = pp[1:2, :]
    px2 = pp[2:3, :]
    py2 = pp[3:4, :]
    area_p = pp[4:5, :]
    pcx = pp[5:6, :]
    pcy = pp[6:7, :]
    inv01w = pp[7:8, :]
    inv01h = pp[8:9, :]
    lpw5 = pp[9:10, :]
    lph5 = pp[10:11, :]

    iw = jnp.maximum(jnp.minimum(tx2, px2) - jnp.maximum(tx1, px1), 0.0)
    ih = jnp.maximum(jnp.minimum(ty2, py2) - jnp.maximum(ty1, py1), 0.0)
    inter = iw * ih
    ov = inter / (area_t + area_p - inter)

    iota_p = lax.broadcasted_iota(jnp.int32, (1, _P_PAD), 1)
    iota_t = lax.broadcasted_iota(jnp.int32, (_T, 1), 0)

    bp_max = jnp.max(ov, axis=1, keepdims=True)
    bp_idx = jnp.min(jnp.where(ov == bp_max, iota_p, _BIG),
                     axis=1, keepdims=True)
    bt_ov = jnp.max(ov, axis=0, keepdims=True)
    bt_idx = jnp.min(jnp.where(ov == bt_ov, iota_t, _BIG),
                     axis=0, keepdims=True)

    eq = bp_idx == iota_p
    forced = jnp.max(jnp.where(eq, iota_t, -1), axis=0, keepdims=True)
    bt_ov = jnp.where(forced >= 0, 2.0, bt_ov)
    bt_idx = jnp.where(forced >= 0, forced, bt_idx)

    onehot = jnp.where(bt_idx == iota_t, 1.0, 0.0)
    m = lax.dot_general(qt, onehot, _DN,
                        preferred_element_type=jnp.float32)
    m_lab = jnp.floor(m[4:5, :] + 0.5)
    conf_f = jnp.where(bt_ov < _THRESH, 0.0, m_lab + 1.0)
    pos = conf_f > 0.5
    posf = pos.astype(jnp.float32)
    num_pos = jnp.sum(posf)

    g_cx = (m[0:1, :] - pcx) * inv01w
    g_cy = (m[1:2, :] - pcy) * inv01h
    g_w = m[2:3, :] - lpw5
    g_h = m[3:4, :] - lph5
    lp = loc_ref[u]
    sl1 = jnp.zeros((1, _P_PAD), jnp.float32)
    for i, g in enumerate((g_cx, g_cy, g_w, g_h)):
        d = jnp.abs(lp[i:i + 1, :] - g)
        sl1 = sl1 + jnp.where(d < 1.0, 0.5 * d * d, d - 0.5)
    loss_l = jnp.sum(sl1 * posf)

    x = conf_ref[u]
    xm = jnp.max(x, axis=0, keepdims=True)
    e = jnp.exp(x - xm)
    ones_r = jnp.ones((1, _NUM_CLASSES), jnp.float32)
    dn_std = (((1,), (0,)), ((), ()))
    s = lax.dot_general(ones_r, e, dn_std,
                        preferred_element_type=jnp.float32)
    lse = xm + jnp.log(s)
    iota_cf = lax.broadcasted_iota(
        jnp.int32, (_NUM_CLASSES, 1), 0).astype(jnp.float32)
    xsel_m = jnp.where(conf_f == iota_cf, x, 0.0)
    x_sel = lax.dot_general(ones_r, xsel_m, dn_std,
                            preferred_element_type=jnp.float32)
    sum_pos_ce = jnp.sum((lse - x_sel) * posf)

    valid = iota_p < _P
    loss_gt = jnp.where(jnp.logical_and(valid, jnp.logical_not(pos)),
                        lse - x[0:1, :], 0.0)

    row16 = lax.broadcasted_iota(jnp.int32, (_B, 1), 0)
    bmask = row16 == bb
    lg_ref[...] = jnp.where(bmask, loss_gt, lg_ref[...])
    np_ref[...] = jnp.where(bmask, num_pos, np_ref[...])

    r = lax.broadcasted_iota(jnp.int32, (8, _LANES), 0)
    c = lax.broadcasted_iota(jnp.int32, (8, _LANES), 1)
    contrib = jnp.where((r == 0) & (c == 0), loss_l, 0.0)
    contrib = contrib + jnp.where((r == 0) & (c == 1), sum_pos_ce, 0.0)
    contrib = contrib + jnp.where((r == 0) & (c == 2), num_pos, 0.0)
    return contrib


def _mbl_kernel(tb_ref, tc_ref, pri_ref, loc_ref, conf_ref, out_ref,
                lg_ref, np_ref):
    b = pl.program_id(0)
    nb = pl.num_programs(0)

    total_contrib = jnp.zeros((8, _LANES), jnp.float32)
    for u in range(_BPS):
        total_contrib = total_contrib + _one_batch(
            _BPS * b + u, u, tb_ref, tc_ref, pri_ref, loc_ref, conf_ref,
            lg_ref, np_ref)

    @pl.when(b == 0)
    def _():
        out_ref[...] = total_contrib

    @pl.when(b > 0)
    def _():
        out_ref[...] = out_ref[...] + total_contrib

    @pl.when(b == nb - 1)
    def _():
        r = lax.broadcasted_iota(jnp.int32, (8, _LANES), 0)
        c = lax.broadcasted_iota(jnp.int32, (8, _LANES), 1)
        lg = lg_ref[...]
        npos = np_ref[...][:, 0:1].astype(jnp.int32)
        kk = jnp.minimum(3 * npos, _P - 1)

        def bs_body(_, carry):
            lo, hi = carry
            mid = lo + (hi - lo) // 2
            mid_f = lax.bitcast_convert_type(mid, jnp.float32)
            cnt = jnp.sum((lg > mid_f).astype(jnp.int32),
                          axis=1, keepdims=True)
            below = cnt < kk
            return (jnp.where(below, lo, mid + 1),
                    jnp.where(below, mid, hi))

        lo0 = jnp.zeros((_B, 1), jnp.int32)
        hi0 = jnp.full((_B, 1), 0x7F800000, jnp.int32)
        _, t_bits = lax.fori_loop(0, 31, bs_body, (lo0, hi0))
        t = lax.bitcast_convert_type(t_bits, jnp.float32)
        gt = lg > t
        cnt_gt = jnp.sum(gt.astype(jnp.int32), axis=1, keepdims=True)
        rows = (jnp.sum(jnp.where(gt, lg, 0.0), axis=1, keepdims=True)
                + (kk - cnt_gt).astype(jnp.float32) * t)
        rows = jnp.where(kk > 0, rows, 0.0)
        topk_total = jnp.sum(rows)
        out_ref[...] = out_ref[...] + jnp.where((r == 0) & (c == 1),
                                                topk_total, 0.0)


@functools.partial(jax.jit, static_argnames=("interpret",))
def kernel(loc_preds, conf_preds, priors, target_boxes, target_classes,
           interpret=False):
    B, P, C = conf_preds.shape
    pad = _P_PAD - P

    tb = jnp.zeros((B, _T, _LANES), jnp.float32).at[:, :, :4].set(target_boxes)
    tc = jnp.zeros((B, _T, _LANES), jnp.float32).at[:, :, 0:1].set(
        target_classes.astype(jnp.float32)[..., None])
    pri_t = jnp.transpose(priors, (1, 0))
    pri_pad = jnp.tile(jnp.array([[-10.0], [-10.0], [1.0], [1.0]],
                                 jnp.float32), (1, pad))
    pri_t = jnp.concatenate([pri_t, pri_pad], axis=1)
    pcx, pcy, pw, ph = (pri_t[i] for i in range(4))
    cx1, cy1 = pcx - pw / 2.0, pcy - ph / 2.0
    cx2, cy2 = pcx + pw / 2.0, pcy + ph / 2.0
    pp = jnp.stack([
        cx1, cy1, cx2, cy2,
        (cx2 - cx1) * (cy2 - cy1), pcx, pcy, 10.0 / pw, 10.0 / ph,
        5.0 * jnp.log(pw), 5.0 * jnp.log(ph),
    ])
    pp = jnp.concatenate(
        [pp, jnp.zeros((5, _P_PAD), jnp.float32)], axis=0)
    loc_t = jnp.pad(jnp.transpose(loc_preds, (0, 2, 1)),
                    ((0, 0), (0, 0), (0, pad)))
    conf_t = jnp.pad(jnp.transpose(conf_preds, (0, 2, 1)),
                     ((0, 0), (0, 0), (0, pad)))

    out = pl.pallas_call(
        _mbl_kernel,
        grid=(B // _BPS,),
        in_specs=[
            pl.BlockSpec((_BPS, _T, _LANES), lambda b: (b, 0, 0)),
            pl.BlockSpec((_BPS, _T, _LANES), lambda b: (b, 0, 0)),
            pl.BlockSpec((16, _P_PAD), lambda b: (0, 0)),
            pl.BlockSpec((_BPS, 4, _P_PAD), lambda b: (b, 0, 0)),
            pl.BlockSpec((_BPS, C, _P_PAD), lambda b: (b, 0, 0)),
        ],
        out_specs=pl.BlockSpec((8, _LANES), lambda b: (0, 0)),
        out_shape=jax.ShapeDtypeStruct((8, _LANES), jnp.float32),
        scratch_shapes=[
            pltpu.VMEM((_B, _P_PAD), jnp.float32),
            pltpu.VMEM((_B, _LANES), jnp.float32),
        ],
        interpret=interpret,
    )(tb, tc, pp, loc_t, conf_t)

    n = out[0, 2]
    return out[0, 0] / n, out[0, 1] / n
